# initial kernel scaffold (unmeasured)
import jax
import jax.numpy as jnp
from jax import lax
from jax.experimental import pallas as pl
from jax.experimental.pallas import tpu as pltpu

N_DEV = 8
B = 2
SQ = 512
SKV_LOC = 512
HQ = 8
DH = 64
DM = 768
DQ = HQ * DH
NEG = -1e9


def _expand(s):
    return jnp.concatenate(
        [jnp.broadcast_to(s[:, :, h : h + 1], (B, SQ, DH)) for h in range(HQ)],
        axis=2,
    )


def kernel(x, Wq, K_ext, V_ext, Wo):
    def body(
        x_ref,
        wq_ref,
        k_ref,
        v_ref,
        wo_ref,
        out_ref,
        comm_m,
        comm_l,
        comm_o,
        send_m,
        recv_m,
        send_l,
        recv_l,
        send_o,
        recv_o,
    ):
        my = lax.axis_index("i")
        left = lax.rem(my + N_DEV - 1, N_DEV)
        right = lax.rem(my + 1, N_DEV)

        barrier = pltpu.get_barrier_semaphore()
        for nbr in (left, right):
            pl.semaphore_signal(
                barrier,
                inc=1,
                device_id=(nbr,),
                device_id_type=pl.DeviceIdType.MESH,
            )
        pl.semaphore_wait(barrier, 2)

        kv0 = my * SKV_LOC
        qb = lax.broadcasted_iota(jnp.int32, (SQ, SKV_LOC), 0) // 64
        kb = (kv0 + lax.broadcasted_iota(jnp.int32, (SQ, SKV_LOC), 1)) // 64
        mask = (qb == kb) | (kb == 0) | ((qb + kb) % 3 == 0)

        for b in range(B):
            q_b = jnp.dot(
                x_ref[b], wq_ref[...], preferred_element_type=jnp.float32
            )
            m_cols = []
            l_cols = []
            for h in range(HQ):
                q_bh = q_b[:, h * DH : (h + 1) * DH]
                k_bh = k_ref[b, :, h, :]
                s = (
                    lax.dot_general(
                        q_bh,
                        k_bh,
                        (((1,), (1,)), ((), ())),
                        preferred_element_type=jnp.float32,
                    )
                    * 0.125
                )
                s = jnp.where(mask, s, NEG)
                m_bh = jnp.max(s, axis=1, keepdims=True)
                w = jnp.exp(s - m_bh)
                l_bh = jnp.sum(w, axis=1, keepdims=True)
                o_bh = jnp.dot(
                    w, v_ref[b, :, h, :], preferred_element_type=jnp.float32
                )
                comm_o[0, b, :, h * DH : (h + 1) * DH] = o_bh
                m_cols.append(m_bh)
                l_cols.append(l_bh)
            comm_m[0, b] = jnp.concatenate(m_cols, axis=1)
            comm_l[0, b] = jnp.concatenate(l_cols, axis=1)

        acc_m = comm_m[0]
        acc_l = comm_l[0]
        acc_o = comm_o[0]

        for h in range(N_DEV - 1):
            rdmas = []
            for buf, ss, rs in (
                (comm_m, send_m, recv_m),
                (comm_l, send_l, recv_l),
                (comm_o, send_o, recv_o),
            ):
                rdma = pltpu.make_async_remote_copy(
                    src_ref=buf.at[h],
                    dst_ref=buf.at[h + 1],
                    send_sem=ss.at[h],
                    recv_sem=rs.at[h],
                    device_id=(right,),
                    device_id_type=pl.DeviceIdType.MESH,
                )
                rdma.start()
                rdmas.append(rdma)
            for rdma in rdmas:
                rdma.wait()

            rm = comm_m[h + 1]
            rl = comm_l[h + 1]
            ro = comm_o[h + 1]
            new_m = jnp.maximum(acc_m, rm)
            sa = jnp.exp(acc_m - new_m)
            sb = jnp.exp(rm - new_m)
            acc_l = acc_l * sa + rl * sb
            acc_o = acc_o * _expand(sa) + ro * _expand(sb)
            acc_m = new_m

        ctx = acc_o / _expand(acc_l)
        for b in range(B):
            out_ref[b] = jnp.dot(
                ctx[b], wo_ref[...], preferred_element_type=jnp.float32
            )

    return pl.pallas_call(
        body,
        out_shape=jax.ShapeDtypeStruct((B, SQ, DM), jnp.float32),
        in_specs=[pl.BlockSpec(memory_space=pltpu.VMEM)] * 5,
        out_specs=pl.BlockSpec(memory_space=pltpu.VMEM),
        scratch_shapes=[
            pltpu.VMEM((N_DEV, B, SQ, HQ), jnp.float32),
            pltpu.VMEM((N_DEV, B, SQ, HQ), jnp.float32),
            pltpu.VMEM((N_DEV, B, SQ, DQ), jnp.float32),
            pltpu.SemaphoreType.DMA((N_DEV - 1,)),
            pltpu.SemaphoreType.DMA((N_DEV - 1,)),
            pltpu.SemaphoreType.DMA((N_DEV - 1,)),
            pltpu.SemaphoreType.DMA((N_DEV - 1,)),
            pltpu.SemaphoreType.DMA((N_DEV - 1,)),
            pltpu.SemaphoreType.DMA((N_DEV - 1,)),
        ],
        compiler_params=pltpu.CompilerParams(collective_id=0),
    )(x, Wq, K_ext, V_ext, Wo)


# baseline (device time: 303027 ns/iter reference)
import jax
import jax.numpy as jnp
from jax import lax
from jax.experimental import pallas as pl
from jax.experimental.pallas import tpu as pltpu

N_DEV = 8
B = 2
SQ = 512
SKV_LOC = 512
HQ = 8
DH = 64
DM = 768
DQ = HQ * DH
NEG = -1e9


def _expand(s):
    return jnp.concatenate(
        [jnp.broadcast_to(s[:, :, h : h + 1], (B, SQ, DH)) for h in range(HQ)],
        axis=2,
    )


def kernel(x, Wq, K_ext, V_ext, Wo):
    def body(
        x_ref,
        wq_ref,
        k_ref,
        v_ref,
        wo_ref,
        out_ref,
        comm_m,
        comm_l,
        comm_o,
        send_m,
        recv_m,
        send_l,
        recv_l,
        send_o,
        recv_o,
    ):
        my = lax.axis_index("i")
        left = lax.rem(my + N_DEV - 1, N_DEV)
        right = lax.rem(my + 1, N_DEV)

        barrier = pltpu.get_barrier_semaphore()
        for nbr in (left, right):
            pl.semaphore_signal(
                barrier,
                inc=1,
                device_id=(nbr,),
                device_id_type=pl.DeviceIdType.MESH,
            )
        pl.semaphore_wait(barrier, 2)

        kv0 = my * SKV_LOC
        qb = lax.broadcasted_iota(jnp.int32, (SQ, SKV_LOC), 0) // 64
        kb = (kv0 + lax.broadcasted_iota(jnp.int32, (SQ, SKV_LOC), 1)) // 64
        mask = (qb == kb) | (kb == 0) | ((qb + kb) % 3 == 0)

        for b in range(B):
            q_b = jnp.dot(
                x_ref[b], wq_ref[...], preferred_element_type=jnp.float32
            )
            m_cols = []
            l_cols = []
            for h in range(HQ):
                q_bh = q_b[:, h * DH : (h + 1) * DH]
                k_bh = k_ref[b, :, h, :]
                s = (
                    lax.dot_general(
                        q_bh,
                        k_bh,
                        (((1,), (1,)), ((), ())),
                        preferred_element_type=jnp.float32,
                    )
                    * 0.125
                )
                s = jnp.where(mask, s, NEG)
                m_bh = jnp.max(s, axis=1, keepdims=True)
                w = jnp.exp(s - m_bh)
                l_bh = jnp.sum(w, axis=1, keepdims=True)
                o_bh = jnp.dot(
                    w, v_ref[b, :, h, :], preferred_element_type=jnp.float32
                )
                comm_o[0, b, :, h * DH : (h + 1) * DH] = o_bh
                m_cols.append(m_bh)
                l_cols.append(l_bh)
            comm_m[0, b] = jnp.concatenate(m_cols, axis=1)
            comm_l[0, b] = jnp.concatenate(l_cols, axis=1)

        acc_m = comm_m[0]
        acc_l = comm_l[0]
        acc_o = comm_o[0]

        for h in range(N_DEV - 1):
            rdmas = []
            for buf, ss, rs in (
                (comm_m, send_m, recv_m),
                (comm_l, send_l, recv_l),
                (comm_o, send_o, recv_o),
            ):
                rdma = pltpu.make_async_remote_copy(
                    src_ref=buf.at[h],
                    dst_ref=buf.at[h + 1],
                    send_sem=ss.at[h],
                    recv_sem=rs.at[h],
                    device_id=(right,),
                    device_id_type=pl.DeviceIdType.MESH,
                )
                rdma.start()
                rdmas.append(rdma)
            for rdma in rdmas:
                rdma.wait()

            rm = comm_m[h + 1]
            rl = comm_l[h + 1]
            ro = comm_o[h + 1]
            new_m = jnp.maximum(acc_m, rm)
            sa = jnp.exp(acc_m - new_m)
            sb = jnp.exp(rm - new_m)
            acc_l = acc_l * sa + rl * sb
            acc_o = acc_o * _expand(sa) + ro * _expand(sb)
            acc_m = new_m

        ctx = acc_o / _expand(acc_l)
        for b in range(B):
            out_ref[b] = jnp.dot(
                ctx[b], wo_ref[...], preferred_element_type=jnp.float32
            )

    return pl.pallas_call(
        body,
        out_shape=jax.ShapeDtypeStruct((B, SQ, DM), jnp.float32),
        in_specs=[pl.BlockSpec(memory_space=pltpu.VMEM)] * 5,
        out_specs=pl.BlockSpec(memory_space=pltpu.VMEM),
        scratch_shapes=[
            pltpu.VMEM((N_DEV, B, SQ, HQ), jnp.float32),
            pltpu.VMEM((N_DEV, B, SQ, HQ), jnp.float32),
            pltpu.VMEM((N_DEV, B, SQ, DQ), jnp.float32),
            pltpu.SemaphoreType.DMA((N_DEV - 1,)),
            pltpu.SemaphoreType.DMA((N_DEV - 1,)),
            pltpu.SemaphoreType.DMA((N_DEV - 1,)),
            pltpu.SemaphoreType.DMA((N_DEV - 1,)),
            pltpu.SemaphoreType.DMA((N_DEV - 1,)),
            pltpu.SemaphoreType.DMA((N_DEV - 1,)),
        ],
        compiler_params=pltpu.CompilerParams(
            collective_id=0, vmem_limit_bytes=100 * 1024 * 1024
        ),
    )(x, Wq, K_ext, V_ext, Wo)


# device time: 90251 ns/iter; 3.3576x vs baseline; 3.3576x over previous
import jax
import jax.numpy as jnp
from jax import lax
from jax.experimental import pallas as pl
from jax.experimental.pallas import tpu as pltpu

N_DEV = 8
B = 2
SQ = 512
SKV_LOC = 512
HQ = 8
DH = 64
DM = 768
DQ = HQ * DH
NEG = -1e9

RS_HALVES = (256, 128, 64)
RS_XORS = (3, 1, 4)
RS_SHIFTS = (1, 0, 2)


def _expand(s, rows):
    return jnp.concatenate(
        [jnp.broadcast_to(s[:, h : h + 1], (rows, DH)) for h in range(HQ)],
        axis=1,
    )


def _combine(keep_s, rec_s, keep_o, rec_o, rows):
    new_o = []
    cols = []
    for b in range(B):
        am = keep_s[:, b * 16 : b * 16 + 8]
        al = keep_s[:, b * 16 + 8 : b * 16 + 16]
        rm = rec_s[:, b * 16 : b * 16 + 8]
        rl = rec_s[:, b * 16 + 8 : b * 16 + 16]
        nm = jnp.maximum(am, rm)
        sa = jnp.exp(am - nm)
        sb = jnp.exp(rm - nm)
        nl = al * sa + rl * sb
        new_o.append(keep_o[b] * _expand(sa, rows) + rec_o[b] * _expand(sb, rows))
        cols += [nm, nl]
    return jnp.concatenate(cols, axis=1), new_o


def kernel(x, Wq, K_ext, V_ext, Wo):
    def body(
        x_ref,
        wq_ref,
        k_ref,
        v_ref,
        wo_ref,
        out_ref,
        ctx_ref,
        so0,
        so1,
        so2,
        ro0,
        ro1,
        ro2,
        ss0,
        ss1,
        ss2,
        rs0,
        rs1,
        rs2,
        sem_so,
        sem_ro,
        sem_ss,
        sem_rs,
        sem_ag_s,
        sem_ag_r,
    ):
        my = lax.axis_index("i")
        partners = [jnp.bitwise_xor(my, RS_XORS[s]) for s in range(3)]

        barrier = pltpu.get_barrier_semaphore()
        for p in partners:
            pl.semaphore_signal(
                barrier,
                inc=1,
                device_id=(p,),
                device_id_type=pl.DeviceIdType.MESH,
            )
        pl.semaphore_wait(barrier, 3)

        kv0 = my * SKV_LOC
        qb = lax.broadcasted_iota(jnp.int32, (SQ, SKV_LOC), 0) // 64
        kb = (kv0 + lax.broadcasted_iota(jnp.int32, (SQ, SKV_LOC), 1)) // 64
        mask = (qb == kb) | (kb == 0) | ((qb + kb) % 3 == 0)

        acc_o = []
        s_cols = []
        for b in range(B):
            q_b = jnp.dot(
                x_ref[b], wq_ref[...], preferred_element_type=jnp.float32
            )
            m_cols = []
            l_cols = []
            o_blocks = []
            for h in range(HQ):
                q_bh = q_b[:, h * DH : (h + 1) * DH]
                k_bh = k_ref[b, :, h, :]
                s = (
                    lax.dot_general(
                        q_bh,
                        k_bh,
                        (((1,), (1,)), ((), ())),
                        preferred_element_type=jnp.float32,
                    )
                    * 0.125
                )
                s = jnp.where(mask, s, NEG)
                m_bh = jnp.max(s, axis=1, keepdims=True)
                w = jnp.exp(s - m_bh)
                l_bh = jnp.sum(w, axis=1, keepdims=True)
                o_blocks.append(
                    jnp.dot(
                        w, v_ref[b, :, h, :], preferred_element_type=jnp.float32
                    )
                )
                m_cols.append(m_bh)
                l_cols.append(l_bh)
            acc_o.append(jnp.concatenate(o_blocks, axis=1))
            s_cols += m_cols + l_cols
        acc_s = jnp.concatenate(s_cols, axis=1)

        send_o_refs = (so0, so1, so2)
        recv_o_refs = (ro0, ro1, ro2)
        send_s_refs = (ss0, ss1, ss2)
        recv_s_refs = (rs0, rs1, rs2)
        for s in range(3):
            half = RS_HALVES[s]
            is_low = ((my >> RS_SHIFTS[s]) & 1) == 0
            keep_o = []
            for b in range(B):
                lo_b = acc_o[b][:half]
                hi_b = acc_o[b][half:]
                keep_o.append(jnp.where(is_low, lo_b, hi_b))
                send_o_refs[s][b] = jnp.where(is_low, hi_b, lo_b)
            keep_s = jnp.where(is_low, acc_s[:half], acc_s[half:])
            send_s_refs[s][...] = jnp.where(is_low, acc_s[half:], acc_s[:half])

            rdma_o = pltpu.make_async_remote_copy(
                src_ref=send_o_refs[s],
                dst_ref=recv_o_refs[s],
                send_sem=sem_so.at[s],
                recv_sem=sem_ro.at[s],
                device_id=(partners[s],),
                device_id_type=pl.DeviceIdType.MESH,
            )
            rdma_s = pltpu.make_async_remote_copy(
                src_ref=send_s_refs[s],
                dst_ref=recv_s_refs[s],
                send_sem=sem_ss.at[s],
                recv_sem=sem_rs.at[s],
                device_id=(partners[s],),
                device_id_type=pl.DeviceIdType.MESH,
            )
            rdma_o.start()
            rdma_s.start()
            rdma_o.wait()
            rdma_s.wait()

            rec_o = [recv_o_refs[s][b] for b in range(B)]
            rec_s = recv_s_refs[s][...]
            acc_s, acc_o = _combine(keep_s, rec_s, keep_o, rec_o, half)

        c = (((my >> 1) & 1) << 2) | ((my & 1) << 1) | ((my >> 2) & 1)
        g0 = c * 64
        for b in range(B):
            l_b = acc_s[:, b * 16 + 8 : b * 16 + 16]
            ctx_ref[b, pl.ds(g0, 64), :] = acc_o[b] / _expand(l_b, 64)

        ag_lens = (64, 128, 256)
        ag_los = (g0, (c & 6) * 64, (c & 4) * 64)
        for s in range(3):
            p = partners[2 - s]
            rdma = pltpu.make_async_remote_copy(
                src_ref=ctx_ref.at[:, pl.ds(ag_los[s], ag_lens[s]), :],
                dst_ref=ctx_ref.at[:, pl.ds(ag_los[s], ag_lens[s]), :],
                send_sem=sem_ag_s.at[s],
                recv_sem=sem_ag_r.at[s],
                device_id=(p,),
                device_id_type=pl.DeviceIdType.MESH,
            )
            rdma.start()
            rdma.wait()

        for b in range(B):
            out_ref[b] = jnp.dot(
                ctx_ref[b], wo_ref[...], preferred_element_type=jnp.float32
            )

    return pl.pallas_call(
        body,
        out_shape=jax.ShapeDtypeStruct((B, SQ, DM), jnp.float32),
        in_specs=[pl.BlockSpec(memory_space=pltpu.VMEM)] * 5,
        out_specs=pl.BlockSpec(memory_space=pltpu.VMEM),
        scratch_shapes=[
            pltpu.VMEM((B, SQ, DQ), jnp.float32),
            pltpu.VMEM((B, 256, DQ), jnp.float32),
            pltpu.VMEM((B, 128, DQ), jnp.float32),
            pltpu.VMEM((B, 64, DQ), jnp.float32),
            pltpu.VMEM((B, 256, DQ), jnp.float32),
            pltpu.VMEM((B, 128, DQ), jnp.float32),
            pltpu.VMEM((B, 64, DQ), jnp.float32),
            pltpu.VMEM((256, 32), jnp.float32),
            pltpu.VMEM((128, 32), jnp.float32),
            pltpu.VMEM((64, 32), jnp.float32),
            pltpu.VMEM((256, 32), jnp.float32),
            pltpu.VMEM((128, 32), jnp.float32),
            pltpu.VMEM((64, 32), jnp.float32),
            pltpu.SemaphoreType.DMA((3,)),
            pltpu.SemaphoreType.DMA((3,)),
            pltpu.SemaphoreType.DMA((3,)),
            pltpu.SemaphoreType.DMA((3,)),
            pltpu.SemaphoreType.DMA((3,)),
            pltpu.SemaphoreType.DMA((3,)),
        ],
        compiler_params=pltpu.CompilerParams(
            collective_id=0, vmem_limit_bytes=100 * 1024 * 1024
        ),
    )(x, Wq, K_ext, V_ext, Wo)


# device time: 73448 ns/iter; 4.1257x vs baseline; 1.2288x over previous
import jax
import jax.numpy as jnp
from jax import lax
from jax.experimental import pallas as pl
from jax.experimental.pallas import tpu as pltpu

N_DEV = 8
B = 2
SQ = 512
SKV_LOC = 512
HQ = 8
DH = 64
DM = 768
DQ = HQ * DH
CHUNK = SQ // N_DEV
NEG = -1e9


def _expand(s, rows):
    return jnp.concatenate(
        [jnp.broadcast_to(s[:, h : h + 1], (rows, DH)) for h in range(HQ)],
        axis=1,
    )


def _combine(acc_s, rec_s, acc_o, rec_o, rows):
    new_o = []
    cols = []
    for b in range(B):
        am = acc_s[:, b * 16 : b * 16 + 8]
        al = acc_s[:, b * 16 + 8 : b * 16 + 16]
        rm = rec_s[:, b * 16 : b * 16 + 8]
        rl = rec_s[:, b * 16 + 8 : b * 16 + 16]
        nm = jnp.maximum(am, rm)
        sa = jnp.exp(am - nm)
        sb = jnp.exp(rm - nm)
        nl = al * sa + rl * sb
        new_o.append(acc_o[b] * _expand(sa, rows) + rec_o[b] * _expand(sb, rows))
        cols += [nm, nl]
    return jnp.concatenate(cols, axis=1), new_o


def kernel(x, Wq, K_ext, V_ext, Wo):
    def body(
        x_ref,
        wq_ref,
        k_ref,
        v_ref,
        wo_ref,
        out_ref,
        o_work,
        s_work,
        recv_o,
        recv_s,
        ctx_ref,
        sem_rs_send,
        sem_rs_recv_o,
        sem_rs_recv_s,
        sem_rs_send_s,
        sem_ag_send,
        sem_ag_recv,
    ):
        my = lax.axis_index("i")

        barrier = pltpu.get_barrier_semaphore()
        for k in range(1, N_DEV):
            pl.semaphore_signal(
                barrier,
                inc=1,
                device_id=(lax.rem(my + k, N_DEV),),
                device_id_type=pl.DeviceIdType.MESH,
            )
        pl.semaphore_wait(barrier, N_DEV - 1)

        kv0 = my * SKV_LOC
        qb = lax.broadcasted_iota(jnp.int32, (SQ, SKV_LOC), 0) // 64
        kb = (kv0 + lax.broadcasted_iota(jnp.int32, (SQ, SKV_LOC), 1)) // 64
        mask = (qb == kb) | (kb == 0) | ((qb + kb) % 3 == 0)

        s_cols = []
        for b in range(B):
            q_b = jnp.dot(
                x_ref[b], wq_ref[...], preferred_element_type=jnp.float32
            )
            m_cols = []
            l_cols = []
            o_blocks = []
            for h in range(HQ):
                q_bh = q_b[:, h * DH : (h + 1) * DH]
                k_bh = k_ref[b, :, h, :]
                s = (
                    lax.dot_general(
                        q_bh,
                        k_bh,
                        (((1,), (1,)), ((), ())),
                        preferred_element_type=jnp.float32,
                    )
                    * 0.125
                )
                s = jnp.where(mask, s, NEG)
                m_bh = jnp.max(s, axis=1, keepdims=True)
                w = jnp.exp(s - m_bh)
                l_bh = jnp.sum(w, axis=1, keepdims=True)
                o_blocks.append(
                    jnp.dot(
                        w, v_ref[b, :, h, :], preferred_element_type=jnp.float32
                    )
                )
                m_cols.append(m_bh)
                l_cols.append(l_bh)
            o_work[b] = jnp.concatenate(o_blocks, axis=1)
            s_cols += m_cols + l_cols
        s_work[...] = jnp.concatenate(s_cols, axis=1)

        rs_rdmas = []
        for k in range(1, N_DEV):
            p = lax.rem(my + k, N_DEV)
            slot = N_DEV - 1 - k
            rdma_o = pltpu.make_async_remote_copy(
                src_ref=o_work.at[:, pl.ds(p * CHUNK, CHUNK), :],
                dst_ref=recv_o.at[slot],
                send_sem=sem_rs_send.at[k - 1],
                recv_sem=sem_rs_recv_o.at[slot],
                device_id=(p,),
                device_id_type=pl.DeviceIdType.MESH,
            )
            rdma_s = pltpu.make_async_remote_copy(
                src_ref=s_work.at[pl.ds(p * CHUNK, CHUNK), :],
                dst_ref=recv_s.at[slot],
                send_sem=sem_rs_send_s.at[k - 1],
                recv_sem=sem_rs_recv_s.at[slot],
                device_id=(p,),
                device_id_type=pl.DeviceIdType.MESH,
            )
            rdma_o.start()
            rdma_s.start()
            rs_rdmas.append((rdma_o, rdma_s))

        acc_o = [o_work[b, pl.ds(my * CHUNK, CHUNK), :] for b in range(B)]
        acc_s = s_work[pl.ds(my * CHUNK, CHUNK), :]

        for rdma_o, rdma_s in rs_rdmas:
            rdma_o.wait()
            rdma_s.wait()
        for j in range(N_DEV - 1):
            rec_o = [recv_o[j, b] for b in range(B)]
            acc_s, acc_o = _combine(acc_s, recv_s[j], acc_o, rec_o, CHUNK)

        for b in range(B):
            l_b = acc_s[:, b * 16 + 8 : b * 16 + 16]
            ctx_ref[b, pl.ds(my * CHUNK, CHUNK), :] = acc_o[b] / _expand(
                l_b, CHUNK
            )

        ag_rdmas = []
        for k in range(1, N_DEV):
            p = lax.rem(my + k, N_DEV)
            rdma = pltpu.make_async_remote_copy(
                src_ref=ctx_ref.at[:, pl.ds(my * CHUNK, CHUNK), :],
                dst_ref=ctx_ref.at[:, pl.ds(my * CHUNK, CHUNK), :],
                send_sem=sem_ag_send.at[k - 1],
                recv_sem=sem_ag_recv.at[N_DEV - 1 - k],
                device_id=(p,),
                device_id_type=pl.DeviceIdType.MESH,
            )
            rdma.start()
            ag_rdmas.append(rdma)
        for rdma in ag_rdmas:
            rdma.wait()

        for b in range(B):
            out_ref[b] = jnp.dot(
                ctx_ref[b], wo_ref[...], preferred_element_type=jnp.float32
            )

    return pl.pallas_call(
        body,
        out_shape=jax.ShapeDtypeStruct((B, SQ, DM), jnp.float32),
        in_specs=[pl.BlockSpec(memory_space=pltpu.VMEM)] * 5,
        out_specs=pl.BlockSpec(memory_space=pltpu.VMEM),
        scratch_shapes=[
            pltpu.VMEM((B, SQ, DQ), jnp.float32),
            pltpu.VMEM((SQ, 32), jnp.float32),
            pltpu.VMEM((N_DEV - 1, B, CHUNK, DQ), jnp.float32),
            pltpu.VMEM((N_DEV - 1, CHUNK, 32), jnp.float32),
            pltpu.VMEM((B, SQ, DQ), jnp.float32),
            pltpu.SemaphoreType.DMA((N_DEV - 1,)),
            pltpu.SemaphoreType.DMA((N_DEV - 1,)),
            pltpu.SemaphoreType.DMA((N_DEV - 1,)),
            pltpu.SemaphoreType.DMA((N_DEV - 1,)),
            pltpu.SemaphoreType.DMA((N_DEV - 1,)),
            pltpu.SemaphoreType.DMA((N_DEV - 1,)),
        ],
        compiler_params=pltpu.CompilerParams(
            collective_id=0, vmem_limit_bytes=100 * 1024 * 1024
        ),
    )(x, Wq, K_ext, V_ext, Wo)


# device time: 56493 ns/iter; 5.3640x vs baseline; 1.3001x over previous
import jax
import jax.numpy as jnp
from jax import lax
from jax.experimental import pallas as pl
from jax.experimental.pallas import tpu as pltpu

N_DEV = 8
B = 2
SQ = 512
SKV_LOC = 512
HQ = 8
DH = 64
DM = 768
DQ = HQ * DH
CHUNK = SQ // N_DEV
NEG = -1e9


def _expand(s, rows):
    return jnp.concatenate(
        [jnp.broadcast_to(s[:, h : h + 1], (rows, DH)) for h in range(HQ)],
        axis=1,
    )


def _combine(acc_s, rec_s, acc_o, rec_o, rows):
    new_o = []
    cols = []
    for b in range(B):
        am = acc_s[:, b * 16 : b * 16 + 8]
        al = acc_s[:, b * 16 + 8 : b * 16 + 16]
        rm = rec_s[:, b * 16 : b * 16 + 8]
        rl = rec_s[:, b * 16 + 8 : b * 16 + 16]
        nm = jnp.maximum(am, rm)
        sa = jnp.exp(am - nm)
        sb = jnp.exp(rm - nm)
        nl = al * sa + rl * sb
        new_o.append(acc_o[b] * _expand(sa, rows) + rec_o[b] * _expand(sb, rows))
        cols += [nm, nl]
    return jnp.concatenate(cols, axis=1), new_o


def kernel(x, Wq, K_ext, V_ext, Wo):
    def body(
        x_ref,
        wq_ref,
        k_ref,
        v_ref,
        wo_ref,
        out_ref,
        o_work,
        s_work,
        recv_o,
        recv_s,
        ctx_ref,
        sem_rs_send,
        sem_rs_recv_o,
        sem_rs_recv_s,
        sem_rs_send_s,
        sem_ag_send,
        sem_ag_recv,
    ):
        my = lax.axis_index("i")

        barrier = pltpu.get_barrier_semaphore()
        for k in range(1, N_DEV):
            pl.semaphore_signal(
                barrier,
                inc=1,
                device_id=(lax.rem(my + k, N_DEV),),
                device_id_type=pl.DeviceIdType.MESH,
            )
        pl.semaphore_wait(barrier, N_DEV - 1)

        kv0 = my * SKV_LOC
        qb = lax.broadcasted_iota(jnp.int32, (SQ, SKV_LOC), 0) // 64
        kb = (kv0 + lax.broadcasted_iota(jnp.int32, (SQ, SKV_LOC), 1)) // 64
        mask = (qb == kb) | (kb == 0) | ((qb + kb) % 3 == 0)

        s_cols = []
        for b in range(B):
            q_b = jnp.dot(
                x_ref[b], wq_ref[...], preferred_element_type=jnp.float32
            )
            m_cols = []
            l_cols = []
            o_blocks = []
            for h in range(HQ):
                q_bh = q_b[:, h * DH : (h + 1) * DH]
                k_bh = k_ref[b, :, h, :]
                s = (
                    lax.dot_general(
                        q_bh,
                        k_bh,
                        (((1,), (1,)), ((), ())),
                        preferred_element_type=jnp.float32,
                    )
                    * 0.125
                )
                s = jnp.where(mask, s, NEG)
                m_bh = jnp.max(s, axis=1, keepdims=True)
                w = jnp.exp(s - m_bh)
                l_bh = jnp.sum(w, axis=1, keepdims=True)
                o_blocks.append(
                    jnp.dot(
                        w, v_ref[b, :, h, :], preferred_element_type=jnp.float32
                    )
                )
                m_cols.append(m_bh)
                l_cols.append(l_bh)
            o_b = jnp.concatenate(o_blocks, axis=1)
            o_work[b] = o_b.astype(jnp.bfloat16)
            s_cols += m_cols + l_cols
        s_all = jnp.concatenate(s_cols, axis=1)
        s_work[...] = s_all

        rs_rdmas = []
        for k in range(1, N_DEV):
            p = lax.rem(my + k, N_DEV)
            slot = N_DEV - 1 - k
            rdma_o = pltpu.make_async_remote_copy(
                src_ref=o_work.at[:, pl.ds(p * CHUNK, CHUNK), :],
                dst_ref=recv_o.at[slot],
                send_sem=sem_rs_send.at[k - 1],
                recv_sem=sem_rs_recv_o.at[slot],
                device_id=(p,),
                device_id_type=pl.DeviceIdType.MESH,
            )
            rdma_s = pltpu.make_async_remote_copy(
                src_ref=s_work.at[pl.ds(p * CHUNK, CHUNK), :],
                dst_ref=recv_s.at[slot],
                send_sem=sem_rs_send_s.at[k - 1],
                recv_sem=sem_rs_recv_s.at[slot],
                device_id=(p,),
                device_id_type=pl.DeviceIdType.MESH,
            )
            rdma_o.start()
            rdma_s.start()
            rs_rdmas.append((rdma_o, rdma_s))

        acc_o = [
            o_work[b, pl.ds(my * CHUNK, CHUNK), :].astype(jnp.float32)
            for b in range(B)
        ]
        acc_s = s_work[pl.ds(my * CHUNK, CHUNK), :]

        for idx, (rdma_o, rdma_s) in enumerate(rs_rdmas):
            rdma_o.wait()
            rdma_s.wait()
            slot = N_DEV - 2 - idx
            rec_o = [recv_o[slot, b].astype(jnp.float32) for b in range(B)]
            acc_s, acc_o = _combine(acc_s, recv_s[slot], acc_o, rec_o, CHUNK)

        ctx_own = []
        for b in range(B):
            l_b = acc_s[:, b * 16 + 8 : b * 16 + 16]
            c_b = acc_o[b] / _expand(l_b, CHUNK)
            ctx_own.append(c_b)
            ctx_ref[b, pl.ds(my * CHUNK, CHUNK), :] = c_b.astype(jnp.bfloat16)

        ag_rdmas = []
        for k in range(1, N_DEV):
            p = lax.rem(my + k, N_DEV)
            rdma = pltpu.make_async_remote_copy(
                src_ref=ctx_ref.at[:, pl.ds(my * CHUNK, CHUNK), :],
                dst_ref=ctx_ref.at[:, pl.ds(my * CHUNK, CHUNK), :],
                send_sem=sem_ag_send.at[k - 1],
                recv_sem=sem_ag_recv.at[N_DEV - 1 - k],
                device_id=(p,),
                device_id_type=pl.DeviceIdType.MESH,
            )
            rdma.start()
            ag_rdmas.append(rdma)

        wo_bf = wo_ref[...].astype(jnp.bfloat16)

        for b in range(B):
            out_ref[b, pl.ds(my * CHUNK, CHUNK), :] = jnp.dot(
                ctx_own[b], wo_ref[...], preferred_element_type=jnp.float32
            )

        for k, rdma in enumerate(ag_rdmas, start=1):
            rdma.wait()
            src = lax.rem(my - k + N_DEV, N_DEV)
            for b in range(B):
                blk = ctx_ref[b, pl.ds(src * CHUNK, CHUNK), :]
                out_ref[b, pl.ds(src * CHUNK, CHUNK), :] = jnp.dot(
                    blk, wo_bf, preferred_element_type=jnp.float32
                )

    return pl.pallas_call(
        body,
        out_shape=jax.ShapeDtypeStruct((B, SQ, DM), jnp.float32),
        in_specs=[pl.BlockSpec(memory_space=pltpu.VMEM)] * 5,
        out_specs=pl.BlockSpec(memory_space=pltpu.VMEM),
        scratch_shapes=[
            pltpu.VMEM((B, SQ, DQ), jnp.bfloat16),
            pltpu.VMEM((SQ, 32), jnp.float32),
            pltpu.VMEM((N_DEV - 1, B, CHUNK, DQ), jnp.bfloat16),
            pltpu.VMEM((N_DEV - 1, CHUNK, 32), jnp.float32),
            pltpu.VMEM((B, SQ, DQ), jnp.bfloat16),
            pltpu.SemaphoreType.DMA((N_DEV - 1,)),
            pltpu.SemaphoreType.DMA((N_DEV - 1,)),
            pltpu.SemaphoreType.DMA((N_DEV - 1,)),
            pltpu.SemaphoreType.DMA((N_DEV - 1,)),
            pltpu.SemaphoreType.DMA((N_DEV - 1,)),
            pltpu.SemaphoreType.DMA((N_DEV - 1,)),
        ],
        compiler_params=pltpu.CompilerParams(
            collective_id=0, vmem_limit_bytes=100 * 1024 * 1024
        ),
    )(x, Wq, K_ext, V_ext, Wo)
